# baseline (speedup 1.0000x reference)
import functools

import jax
import jax.numpy as jnp
from jax import lax
from jax.experimental import pallas as pl
from jax.experimental.pallas import tpu as pltpu
from jax.experimental.pallas import tpu_sc as plsc

VOCAB = 1000
D_MODEL = 128
BIGRAM_VOCAB = 1000000
BIGRAM_DIM = 64
B, S = 1024, 20
N = B * S
MOD = BIGRAM_VOCAB - 1
PAIRS = BIGRAM_VOCAB // 2
PAIR_DIM = 2 * BIGRAM_DIM

CHUNK = 128
LANES = 16
PAD = 8


def _sc_gather(tokens_flat, tok_emb, big2):
    info = plsc.get_sparse_core_info()
    nc, ns = info.num_cores, info.num_subcores
    nw = nc * ns
    per_w = N // nw
    nch = per_w // CHUNK

    mesh = plsc.VectorSubcoreMesh(core_axis_name="c", subcore_axis_name="s")

    @functools.partial(
        pl.kernel,
        mesh=mesh,
        out_type=[
            jax.ShapeDtypeStruct((N, D_MODEL), jnp.float32),
            jax.ShapeDtypeStruct((N, PAIR_DIM), jnp.float32),
        ],
        scratch_types=[
            pltpu.VMEM((PAD + per_w,), jnp.int32),
            pltpu.VMEM((per_w,), jnp.int32),
            pltpu.VMEM((per_w,), jnp.int32),
            pltpu.VMEM((2, CHUNK, D_MODEL), jnp.float32),
            pltpu.VMEM((per_w, PAIR_DIM), jnp.float32),
            pltpu.SemaphoreType.DMA,
            pltpu.SemaphoreType.DMA,
        ],
    )
    def k(tok_hbm, temb_hbm, bemb_hbm, xtok_hbm, hbp_hbm,
          tokv, pidx, poff, trows, bpair, tsem, bsem):
        wid = lax.axis_index("s") * nc + lax.axis_index("c")
        base = wid * per_w

        pltpu.sync_copy(tok_hbm.at[pl.ds(base, per_w)],
                        tokv.at[pl.ds(PAD, per_w)])

        lane = lax.broadcasted_iota(jnp.int32, (LANES,), 0)
        for v in range(per_w // LANES):
            j0 = v * LANES
            curr = tokv[pl.ds(PAD + j0, LANES)]
            prev = tokv[pl.ds(PAD + j0 - 1, LANES)]
            h = lax.rem(lax.bitwise_xor(curr * 36313, prev * 27191),
                        jnp.int32(MOD))
            s = lax.rem(lane + jnp.int32(j0 % S), jnp.int32(S))
            h = jnp.where(s == 0, jnp.int32(MOD), h)
            pidx[pl.ds(j0, LANES)] = lax.shift_right_logical(h, 1)
            poff[pl.ds(j0, LANES)] = (
                jnp.int32(1) - lax.bitwise_and(h, jnp.int32(1))) * BIGRAM_DIM

        cps = []
        for c in range(nch):
            sl = pl.ds(c * CHUNK, CHUNK)
            t_cp = pltpu.make_async_copy(
                temb_hbm.at[tokv.at[pl.ds(PAD + c * CHUNK, CHUNK)]],
                trows.at[c % 2], tsem)
            t_cp.start()
            b_cp = pltpu.make_async_copy(
                bemb_hbm.at[pidx.at[sl]], bpair.at[sl], bsem)
            b_cp.start()
            cps.append((t_cp, b_cp, c))
            if c > 0:
                pt, _, pc = cps[c - 1]
                pt.wait()
                out_sl = pl.ds(base + pc * CHUNK, CHUNK)
                pltpu.sync_copy(trows.at[pc % 2], xtok_hbm.at[out_sl])
        lt, _, lc = cps[nch - 1]
        lt.wait()
        pltpu.sync_copy(trows.at[lc % 2],
                        xtok_hbm.at[pl.ds(base + lc * CHUNK, CHUNK)])
        for _, b_cp, _ in cps:
            b_cp.wait()

        zv = jnp.zeros((LANES,), jnp.float32)

        def mask_row(j, carry):
            off = poff[pl.ds(j, LANES)][0]
            bpair[j, pl.ds(off, LANES)] = zv
            bpair[j, pl.ds(off + LANES, LANES)] = zv
            bpair[j, pl.ds(off + 2 * LANES, LANES)] = zv
            bpair[j, pl.ds(off + 3 * LANES, LANES)] = zv
            return carry

        lax.fori_loop(0, per_w, mask_row, 0)

        pltpu.sync_copy(bpair, hbp_hbm.at[pl.ds(base, per_w)])

    return k(tokens_flat, tok_emb, big2)


BATCH_BLK = 64
BLK = BATCH_BLK * S


def _tc_body(xtok_ref, hbp_ref, emb_ref, pw2_ref, bs_ref, g_ref,
             ns_ref, out_ref):
    row = lax.broadcasted_iota(jnp.int32, (BLK, 1), 0)
    s0 = lax.rem(row, S) == 0
    hbp = lax.dot_general(hbp_ref[...], pw2_ref[...],
                          (((1,), (0,)), ((), ())),
                          preferred_element_type=jnp.float32)
    x = xtok_ref[...] + hbp * bs_ref[0, 0]
    g = jax.nn.sigmoid(g_ref[...])
    xs = jnp.concatenate(
        [jnp.zeros((1, D_MODEL), jnp.float32), x[:-1, :]], axis=0)
    xprev = jnp.where(s0, 0.0, xs)
    x = (1.0 - g) * x + g * xprev
    ms = jnp.mean(x * x, axis=1, keepdims=True)
    xn = x * lax.rsqrt(ms + 1e-6) * ns_ref[...]
    logits = lax.dot_general(xn, emb_ref[...],
                             (((1,), (1,)), ((), ())),
                             preferred_element_type=jnp.float32)
    out_ref[...] = logits.reshape(BATCH_BLK, S, VOCAB)


def _tc_dense(x_tok, hbp, tok_emb, pw2, big_scale, gate, norm_scale):
    grid = (B // BATCH_BLK,)
    return pl.pallas_call(
        _tc_body,
        grid=grid,
        in_specs=[
            pl.BlockSpec((BLK, D_MODEL), lambda i: (i, 0)),
            pl.BlockSpec((BLK, PAIR_DIM), lambda i: (i, 0)),
            pl.BlockSpec((VOCAB, D_MODEL), lambda i: (0, 0)),
            pl.BlockSpec((PAIR_DIM, D_MODEL), lambda i: (0, 0)),
            pl.BlockSpec((1, 1), lambda i: (0, 0)),
            pl.BlockSpec((1, D_MODEL), lambda i: (0, 0)),
            pl.BlockSpec((1, D_MODEL), lambda i: (0, 0)),
        ],
        out_specs=pl.BlockSpec((BATCH_BLK, S, VOCAB), lambda i: (i, 0, 0)),
        out_shape=jax.ShapeDtypeStruct((B, S, VOCAB), jnp.float32),
    )(x_tok, hbp, tok_emb, pw2, big_scale, gate, norm_scale)


def kernel(tokens, tok_emb, big_emb, proj_w, big_scale, gate, norm_scale):
    tokens_flat = tokens.reshape(-1).astype(jnp.int32)
    big2 = big_emb.reshape(PAIRS, PAIR_DIM)
    x_tok, hbp = _sc_gather(tokens_flat, tok_emb, big2)
    pw2 = jnp.concatenate([proj_w.T, proj_w.T], axis=0)
    return _tc_dense(x_tok, hbp, tok_emb, pw2,
                     big_scale.reshape(1, 1).astype(jnp.float32),
                     gate.reshape(1, D_MODEL),
                     norm_scale.reshape(1, D_MODEL))


# speedup vs baseline: 1.0415x; 1.0415x over previous
import functools

import jax
import jax.numpy as jnp
from jax import lax
from jax.experimental import pallas as pl
from jax.experimental.pallas import tpu as pltpu
from jax.experimental.pallas import tpu_sc as plsc

VOCAB = 1000
D_MODEL = 128
BIGRAM_VOCAB = 1000000
BIGRAM_DIM = 64
B, S = 1024, 20
N = B * S
MOD = BIGRAM_VOCAB - 1

CHUNK = 128
LANES = 16
PAD = 8


def _sc_gather(tokens_flat, tok_emb, big_emb):
    info = plsc.get_sparse_core_info()
    nc, ns = info.num_cores, info.num_subcores
    nw = nc * ns
    per_w = N // nw
    nch = per_w // CHUNK

    mesh = plsc.VectorSubcoreMesh(core_axis_name="c", subcore_axis_name="s")

    @functools.partial(
        pl.kernel,
        mesh=mesh,
        compiler_params=pltpu.CompilerParams(use_tc_tiling_on_sc=False),
        out_type=[
            jax.ShapeDtypeStruct((N, D_MODEL), jnp.float32),
            jax.ShapeDtypeStruct((N, BIGRAM_DIM), jnp.float32),
        ],
        scratch_types=[
            pltpu.VMEM((PAD + per_w,), jnp.int32),
            pltpu.VMEM((per_w,), jnp.int32),
            pltpu.VMEM((per_w, D_MODEL), jnp.float32),
            pltpu.VMEM((per_w, BIGRAM_DIM), jnp.float32),
            pltpu.SemaphoreType.DMA,
            pltpu.SemaphoreType.DMA,
        ],
    )
    def k(tok_hbm, temb_hbm, bemb_hbm, xtok_hbm, hb_hbm,
          tokv, bidx, trows, brows, tsem, bsem):
        wid = lax.axis_index("s") * nc + lax.axis_index("c")
        base = wid * per_w

        pltpu.sync_copy(tok_hbm.at[pl.ds(base, per_w)],
                        tokv.at[pl.ds(PAD, per_w)])

        lane = lax.broadcasted_iota(jnp.int32, (LANES,), 0)
        for v in range(per_w // LANES):
            j0 = v * LANES
            curr = tokv[pl.ds(PAD + j0, LANES)]
            prev = tokv[pl.ds(PAD + j0 - 1, LANES)]
            h = lax.rem(lax.bitwise_xor(curr * 36313, prev * 27191),
                        jnp.int32(MOD))
            s = lax.rem(lane + jnp.int32(j0 % S), jnp.int32(S))
            bidx[pl.ds(j0, LANES)] = jnp.where(s == 0, jnp.int32(MOD), h)

        cps = []
        for c in range(nch):
            sl = pl.ds(c * CHUNK, CHUNK)
            t_cp = pltpu.make_async_copy(
                temb_hbm.at[tokv.at[pl.ds(PAD + c * CHUNK, CHUNK)]],
                trows.at[sl], tsem)
            t_cp.start()
            b_cp = pltpu.make_async_copy(
                bemb_hbm.at[bidx.at[sl]], brows.at[sl], bsem)
            b_cp.start()
            cps.append((t_cp, b_cp))
        for t_cp, b_cp in cps:
            t_cp.wait()
            b_cp.wait()
        pltpu.sync_copy(trows, xtok_hbm.at[pl.ds(base, per_w)])
        pltpu.sync_copy(brows, hb_hbm.at[pl.ds(base, per_w)])

    return k(tokens_flat, tok_emb, big_emb)


BATCH_BLK = 64
BLK = BATCH_BLK * S


def _tc_body(xtok_ref, hb_ref, emb_ref, pw_ref, bs_ref, g_ref,
             ns_ref, out_ref):
    row = lax.broadcasted_iota(jnp.int32, (BLK, 1), 0)
    s0 = lax.rem(row, S) == 0
    hbp = lax.dot_general(hb_ref[...], pw_ref[...],
                          (((1,), (1,)), ((), ())),
                          preferred_element_type=jnp.float32)
    x = xtok_ref[...] + hbp * bs_ref[0, 0]
    g = jax.nn.sigmoid(g_ref[...])
    xs = jnp.concatenate(
        [jnp.zeros((1, D_MODEL), jnp.float32), x[:-1, :]], axis=0)
    xprev = jnp.where(s0, 0.0, xs)
    x = (1.0 - g) * x + g * xprev
    ms = jnp.mean(x * x, axis=1, keepdims=True)
    xn = x * lax.rsqrt(ms + 1e-6) * ns_ref[...]
    logits = lax.dot_general(xn, emb_ref[...],
                             (((1,), (1,)), ((), ())),
                             preferred_element_type=jnp.float32)
    out_ref[...] = logits.reshape(BATCH_BLK, S, VOCAB)


def _tc_dense(x_tok, hb, tok_emb, proj_w, big_scale, gate, norm_scale):
    grid = (B // BATCH_BLK,)
    return pl.pallas_call(
        _tc_body,
        grid=grid,
        in_specs=[
            pl.BlockSpec((BLK, D_MODEL), lambda i: (i, 0)),
            pl.BlockSpec((BLK, BIGRAM_DIM), lambda i: (i, 0)),
            pl.BlockSpec((VOCAB, D_MODEL), lambda i: (0, 0)),
            pl.BlockSpec((D_MODEL, BIGRAM_DIM), lambda i: (0, 0)),
            pl.BlockSpec((1, 1), lambda i: (0, 0)),
            pl.BlockSpec((1, D_MODEL), lambda i: (0, 0)),
            pl.BlockSpec((1, D_MODEL), lambda i: (0, 0)),
        ],
        out_specs=pl.BlockSpec((BATCH_BLK, S, VOCAB), lambda i: (i, 0, 0)),
        out_shape=jax.ShapeDtypeStruct((B, S, VOCAB), jnp.float32),
    )(x_tok, hb, tok_emb, proj_w, big_scale, gate, norm_scale)


def kernel(tokens, tok_emb, big_emb, proj_w, big_scale, gate, norm_scale):
    tokens_flat = tokens.reshape(-1).astype(jnp.int32)
    x_tok, hb = _sc_gather(tokens_flat, tok_emb, big_emb)
    return _tc_dense(x_tok, hb, tok_emb, proj_w,
                     big_scale.reshape(1, 1).astype(jnp.float32),
                     gate.reshape(1, D_MODEL),
                     norm_scale.reshape(1, D_MODEL))


# speedup vs baseline: 1.7840x; 1.7129x over previous
import functools

import jax
import jax.numpy as jnp
from jax import lax
from jax.experimental import pallas as pl
from jax.experimental.pallas import tpu as pltpu
from jax.experimental.pallas import tpu_sc as plsc

VOCAB = 1000
D_MODEL = 128
BIGRAM_VOCAB = 1000000
BIGRAM_DIM = 64
B, S = 1024, 20
N = B * S
MOD = BIGRAM_VOCAB - 1

CHUNK = 128
LANES = 16


def _sc_gather(tokens_t, tok_emb, big_emb):
    info = plsc.get_sparse_core_info()
    nc, ns = info.num_cores, info.num_subcores
    nw = nc * ns
    bw = B // nw
    per_w = bw * S
    nch = per_w // CHUNK

    mesh = plsc.VectorSubcoreMesh(core_axis_name="c", subcore_axis_name="s")

    @functools.partial(
        pl.kernel,
        mesh=mesh,
        out_type=[
            jax.ShapeDtypeStruct((N, D_MODEL), jnp.float32),
            jax.ShapeDtypeStruct((N, BIGRAM_DIM), jnp.float32),
        ],
        scratch_types=[
            pltpu.VMEM((S * bw,), jnp.int32),
            pltpu.VMEM((per_w,), jnp.int32),
            pltpu.VMEM((per_w,), jnp.int32),
            pltpu.VMEM((2, CHUNK, D_MODEL), jnp.float32),
            pltpu.VMEM((per_w, BIGRAM_DIM), jnp.float32),
            pltpu.SemaphoreType.DMA,
            pltpu.SemaphoreType.DMA,
        ],
    )
    def k(tokt_hbm, temb_hbm, bemb_hbm, xtok_hbm, hb_hbm,
          tokv, tidx, bidx, trows, brows, tsem, bsem):
        wid = lax.axis_index("s") * nc + lax.axis_index("c")
        b0 = wid * bw

        for s in range(S):
            pltpu.sync_copy(tokt_hbm.at[s, pl.ds(b0, bw)],
                            tokv.at[pl.ds(s * bw, bw)])

        for s in range(S):
            for v in range(bw // LANES):
                j0 = s * bw + v * LANES
                curr = tokv[pl.ds(j0, LANES)]
                tidx[pl.ds(j0, LANES)] = curr
                if s == 0:
                    bidx[pl.ds(j0, LANES)] = jnp.full((LANES,), MOD, jnp.int32)
                else:
                    prev = tokv[pl.ds(j0 - bw, LANES)]
                    h = lax.rem(lax.bitwise_xor(curr * 36313, prev * 27191),
                                jnp.int32(MOD))
                    bidx[pl.ds(j0, LANES)] = h

        def fire(j, carry):
            idx = bidx[pl.ds(j, LANES)][0]
            pltpu.make_async_copy(bemb_hbm.at[pl.ds(idx, 1)],
                                  brows.at[pl.ds(j, 1)], bsem).start()
            return carry

        lax.fori_loop(0, per_w, fire, 0)

        def flush_tok(c):
            for g in range(CHUNK // bw):
                s = c * (CHUNK // bw) + g
                pltpu.sync_copy(trows.at[c % 2, pl.ds(g * bw, bw)],
                                xtok_hbm.at[pl.ds(s * B + b0, bw)])

        cps = []
        for c in range(nch):
            sl = pl.ds(c * CHUNK, CHUNK)
            t_cp = pltpu.make_async_copy(
                temb_hbm.at[tidx.at[sl]], trows.at[c % 2], tsem)
            t_cp.start()
            cps.append(t_cp)
            if c > 0:
                cps[c - 1].wait()
                flush_tok(c - 1)
        cps[nch - 1].wait()
        flush_tok(nch - 1)
        pltpu.make_async_copy(bemb_hbm.at[pl.ds(0, per_w)],
                              brows.at[pl.ds(0, per_w)], bsem).wait()

        for s in range(S):
            pltpu.sync_copy(brows.at[pl.ds(s * bw, bw)],
                            hb_hbm.at[pl.ds(s * B + b0, bw)])

    return k(tokens_t, tok_emb, big_emb)


BATCH_BLK = 128


def _tc_body(xtok_ref, hb_ref, emb_ref, pw_ref, bs_ref, g_ref,
             ns_ref, out_ref):
    hbp = lax.dot_general(hb_ref[...], pw_ref[...],
                          (((2,), (1,)), ((), ())),
                          preferred_element_type=jnp.float32)
    x = xtok_ref[...] + hbp * bs_ref[0, 0]
    g = jax.nn.sigmoid(g_ref[...])
    xprev = jnp.concatenate(
        [jnp.zeros((1, BATCH_BLK, D_MODEL), jnp.float32), x[:-1]], axis=0)
    x = (1.0 - g) * x + g * xprev
    ms = jnp.mean(x * x, axis=2, keepdims=True)
    xn = x * lax.rsqrt(ms + 1e-6) * ns_ref[...]
    for s in range(S):
        lt = lax.dot_general(emb_ref[...], xn[s],
                             (((1,), (1,)), ((), ())),
                             preferred_element_type=jnp.float32)
        out_ref[s] = lt


def _tc_dense(x_tok3, hb3, tok_emb, proj_w, big_scale, gate, norm_scale):
    grid = (B // BATCH_BLK,)
    return pl.pallas_call(
        _tc_body,
        grid=grid,
        in_specs=[
            pl.BlockSpec((S, BATCH_BLK, D_MODEL), lambda i: (0, i, 0)),
            pl.BlockSpec((S, BATCH_BLK, BIGRAM_DIM), lambda i: (0, i, 0)),
            pl.BlockSpec((VOCAB, D_MODEL), lambda i: (0, 0)),
            pl.BlockSpec((D_MODEL, BIGRAM_DIM), lambda i: (0, 0)),
            pl.BlockSpec((1, 1), lambda i: (0, 0)),
            pl.BlockSpec((1, 1, D_MODEL), lambda i: (0, 0, 0)),
            pl.BlockSpec((1, 1, D_MODEL), lambda i: (0, 0, 0)),
        ],
        out_specs=pl.BlockSpec((S, VOCAB, BATCH_BLK), lambda i: (0, 0, i)),
        out_shape=jax.ShapeDtypeStruct((S, VOCAB, B), jnp.float32),
    )(x_tok3, hb3, tok_emb, proj_w, big_scale, gate, norm_scale)


def kernel(tokens, tok_emb, big_emb, proj_w, big_scale, gate, norm_scale):
    tokens_t = tokens.T.astype(jnp.int32)
    x_tok, hb = _sc_gather(tokens_t, tok_emb, big_emb)
    x_tok3 = x_tok.reshape(S, B, D_MODEL)
    hb3 = hb.reshape(S, B, BIGRAM_DIM)
    out_t = _tc_dense(x_tok3, hb3, tok_emb, proj_w,
                      big_scale.reshape(1, 1).astype(jnp.float32),
                      gate.reshape(1, 1, D_MODEL),
                      norm_scale.reshape(1, 1, D_MODEL))
    return out_t.transpose(2, 0, 1)


# speedup vs baseline: 1.8387x; 1.0307x over previous
import functools

import jax
import jax.numpy as jnp
from jax import lax
from jax.experimental import pallas as pl
from jax.experimental.pallas import tpu as pltpu
from jax.experimental.pallas import tpu_sc as plsc

VOCAB = 1000
D_MODEL = 128
BIGRAM_VOCAB = 1000000
BIGRAM_DIM = 64
B, S = 1024, 20
N = B * S
MOD = BIGRAM_VOCAB - 1

CHUNK = 128
LANES = 16

_INFO = plsc.get_sparse_core_info()
_NW = _INFO.num_cores * _INFO.num_subcores
BW = B // _NW
PER_W = BW * S
_MESH = dict(core_axis_name="c", subcore_axis_name="s")


def _stage_tokens(tokt_hbm, tokv, b0):
    for s in range(S):
        pltpu.sync_copy(tokt_hbm.at[s, pl.ds(b0, BW)],
                        tokv.at[pl.ds(s * BW, BW)])


def _sc_tok(tokens_t, tok_emb):
    nch = PER_W // CHUNK

    @functools.partial(
        pl.kernel,
        mesh=plsc.VectorSubcoreMesh(**_MESH),
        out_type=jax.ShapeDtypeStruct((N, D_MODEL), jnp.float32),
        scratch_types=[
            pltpu.VMEM((S * BW,), jnp.int32),
            pltpu.VMEM((2, CHUNK, D_MODEL), jnp.float32),
            pltpu.SemaphoreType.DMA,
        ],
    )
    def k1(tokt_hbm, temb_hbm, xtok_hbm, tokv, trows, tsem):
        wid = lax.axis_index("s") * _INFO.num_cores + lax.axis_index("c")
        b0 = wid * BW
        _stage_tokens(tokt_hbm, tokv, b0)

        def flush(c):
            for g in range(CHUNK // BW):
                s = c * (CHUNK // BW) + g
                pltpu.sync_copy(trows.at[c % 2, pl.ds(g * BW, BW)],
                                xtok_hbm.at[pl.ds(s * B + b0, BW)])

        cps = []
        for c in range(nch):
            t_cp = pltpu.make_async_copy(
                temb_hbm.at[tokv.at[pl.ds(c * CHUNK, CHUNK)]],
                trows.at[c % 2], tsem)
            t_cp.start()
            cps.append(t_cp)
            if c > 0:
                cps[c - 1].wait()
                flush(c - 1)
        cps[nch - 1].wait()
        flush(nch - 1)

    return k1(tokens_t, tok_emb)


def _sc_big(tokens_t, big_emb):
    @functools.partial(
        pl.kernel,
        mesh=plsc.VectorSubcoreMesh(**_MESH),
        out_type=jax.ShapeDtypeStruct((N, BIGRAM_DIM), jnp.float32),
        scratch_types=[
            pltpu.VMEM((S * BW,), jnp.int32),
            pltpu.VMEM((PER_W,), jnp.int32),
            pltpu.VMEM((PER_W, BIGRAM_DIM), jnp.float32),
            pltpu.SemaphoreType.DMA,
        ],
    )
    def k2(tokt_hbm, bemb_hbm, hb_hbm, tokv, bidx, brows, bsem):
        wid = lax.axis_index("s") * _INFO.num_cores + lax.axis_index("c")
        b0 = wid * BW
        _stage_tokens(tokt_hbm, tokv, b0)

        for s in range(1, S):
            for v in range(BW // LANES):
                j0 = s * BW + v * LANES
                curr = tokv[pl.ds(j0, LANES)]
                prev = tokv[pl.ds(j0 - BW, LANES)]
                h = lax.rem(lax.bitwise_xor(curr * 36313, prev * 27191),
                            jnp.int32(MOD))
                bidx[pl.ds(j0, LANES)] = h

        for r in range(8):
            pltpu.make_async_copy(bemb_hbm.at[pl.ds(MOD, 1)],
                                  brows.at[pl.ds(r, 1)], bsem).start()

        def fire(j, carry):
            idx = bidx[pl.ds(j, LANES)][0]
            pltpu.make_async_copy(bemb_hbm.at[pl.ds(idx, 1)],
                                  brows.at[pl.ds(j, 1)], bsem).start()
            return carry

        lax.fori_loop(BW, PER_W, fire, 0)
        pltpu.make_async_copy(bemb_hbm.at[pl.ds(0, PER_W - BW + 8)],
                              brows.at[pl.ds(0, PER_W - BW + 8)], bsem).wait()

        for r in range(8, BW):
            for v in range(BIGRAM_DIM // LANES):
                brows[r, pl.ds(v * LANES, LANES)] = (
                    brows[0, pl.ds(v * LANES, LANES)])

        for s in range(S):
            pltpu.sync_copy(brows.at[pl.ds(s * BW, BW)],
                            hb_hbm.at[pl.ds(s * B + b0, BW)])

    return k2(tokens_t, big_emb)


BATCH_BLK = 128


def _tc_body(xtok_ref, hb_ref, emb_ref, pw_ref, bs_ref, g_ref,
             ns_ref, out_ref):
    hbp = lax.dot_general(hb_ref[...], pw_ref[...],
                          (((2,), (1,)), ((), ())),
                          preferred_element_type=jnp.float32)
    x = xtok_ref[...] + hbp * bs_ref[0, 0]
    g = jax.nn.sigmoid(g_ref[...])
    xprev = jnp.concatenate(
        [jnp.zeros((1, BATCH_BLK, D_MODEL), jnp.float32), x[:-1]], axis=0)
    x = (1.0 - g) * x + g * xprev
    ms = jnp.mean(x * x, axis=2, keepdims=True)
    xn = x * lax.rsqrt(ms + 1e-6) * ns_ref[...]
    for s in range(S):
        lt = lax.dot_general(emb_ref[...], xn[s],
                             (((1,), (1,)), ((), ())),
                             preferred_element_type=jnp.float32)
        out_ref[s] = lt


def _tc_dense(x_tok3, hb3, tok_emb, proj_w, big_scale, gate, norm_scale):
    grid = (B // BATCH_BLK,)
    return pl.pallas_call(
        _tc_body,
        grid=grid,
        in_specs=[
            pl.BlockSpec((S, BATCH_BLK, D_MODEL), lambda i: (0, i, 0)),
            pl.BlockSpec((S, BATCH_BLK, BIGRAM_DIM), lambda i: (0, i, 0)),
            pl.BlockSpec((VOCAB, D_MODEL), lambda i: (0, 0)),
            pl.BlockSpec((D_MODEL, BIGRAM_DIM), lambda i: (0, 0)),
            pl.BlockSpec((1, 1), lambda i: (0, 0)),
            pl.BlockSpec((1, 1, D_MODEL), lambda i: (0, 0, 0)),
            pl.BlockSpec((1, 1, D_MODEL), lambda i: (0, 0, 0)),
        ],
        out_specs=pl.BlockSpec((S, VOCAB, BATCH_BLK), lambda i: (0, 0, i)),
        out_shape=jax.ShapeDtypeStruct((S, VOCAB, B), jnp.float32),
    )(x_tok3, hb3, tok_emb, proj_w, big_scale, gate, norm_scale)


def kernel(tokens, tok_emb, big_emb, proj_w, big_scale, gate, norm_scale):
    tokens_t = tokens.T.astype(jnp.int32)
    x_tok = _sc_tok(tokens_t, tok_emb)
    hb = _sc_big(tokens_t, big_emb)
    x_tok3 = x_tok.reshape(S, B, D_MODEL)
    hb3 = hb.reshape(S, B, BIGRAM_DIM)
    out_t = _tc_dense(x_tok3, hb3, tok_emb, proj_w,
                      big_scale.reshape(1, 1).astype(jnp.float32),
                      gate.reshape(1, 1, D_MODEL),
                      norm_scale.reshape(1, 1, D_MODEL))
    return out_t.transpose(2, 0, 1)


# speedup vs baseline: 1.8805x; 1.0227x over previous
import functools

import jax
import jax.numpy as jnp
from jax import lax
from jax.experimental import pallas as pl
from jax.experimental.pallas import tpu as pltpu
from jax.experimental.pallas import tpu_sc as plsc

VOCAB = 1000
D_MODEL = 128
BIGRAM_VOCAB = 1000000
BIGRAM_DIM = 64
B, S = 1024, 20
N = B * S
MOD = BIGRAM_VOCAB - 1

CHUNK = 128
LANES = 16

_INFO = plsc.get_sparse_core_info()
_NW = _INFO.num_cores * _INFO.num_subcores
BW = B // _NW
PER_W = BW * S
_MESH = dict(core_axis_name="c", subcore_axis_name="s")


def _stage_tokens(tokt_hbm, tokv, b0):
    for s in range(S):
        pltpu.sync_copy(tokt_hbm.at[s, pl.ds(b0, BW)],
                        tokv.at[pl.ds(s * BW, BW)])


def _sc_big(tokens_t, big_emb):
    @functools.partial(
        pl.kernel,
        mesh=plsc.VectorSubcoreMesh(**_MESH),
        out_type=jax.ShapeDtypeStruct((N, BIGRAM_DIM), jnp.float32),
        scratch_types=[
            pltpu.VMEM((S * BW,), jnp.int32),
            pltpu.VMEM((PER_W,), jnp.int32),
            pltpu.VMEM((PER_W, BIGRAM_DIM), jnp.float32),
            pltpu.SemaphoreType.DMA,
        ],
    )
    def k2(tokt_hbm, bemb_hbm, hb_hbm, tokv, bidx, brows, bsem):
        wid = lax.axis_index("s") * _INFO.num_cores + lax.axis_index("c")
        b0 = wid * BW
        _stage_tokens(tokt_hbm, tokv, b0)

        for s in range(1, S):
            for v in range(BW // LANES):
                j0 = s * BW + v * LANES
                curr = tokv[pl.ds(j0, LANES)]
                prev = tokv[pl.ds(j0 - BW, LANES)]
                h = lax.rem(lax.bitwise_xor(curr * 36313, prev * 27191),
                            jnp.int32(MOD))
                bidx[pl.ds(j0, LANES)] = h

        for r in range(8):
            pltpu.make_async_copy(bemb_hbm.at[pl.ds(MOD, 1)],
                                  brows.at[pl.ds(r, 1)], bsem).start()

        def fire(j, carry):
            idx = bidx[pl.ds(j, LANES)][0]
            pltpu.make_async_copy(bemb_hbm.at[pl.ds(idx, 1)],
                                  brows.at[pl.ds(j, 1)], bsem).start()
            return carry

        lax.fori_loop(BW, PER_W, fire, 0)
        pltpu.make_async_copy(bemb_hbm.at[pl.ds(0, PER_W - BW + 8)],
                              brows.at[pl.ds(0, PER_W - BW + 8)], bsem).wait()

        for r in range(8, BW):
            for v in range(BIGRAM_DIM // LANES):
                brows[r, pl.ds(v * LANES, LANES)] = (
                    brows[0, pl.ds(v * LANES, LANES)])

        for s in range(S):
            pltpu.sync_copy(brows.at[pl.ds(s * BW, BW)],
                            hb_hbm.at[pl.ds(s * B + b0, BW)])

    return k2(tokens_t, big_emb)


BATCH_BLK = 128


def _tc_body(tok_ref, hb_ref, emb_ref, pw_ref, bs_ref, g_ref,
             ns_ref, out_ref):
    t3 = tok_ref[...]
    oh = (lax.broadcasted_iota(jnp.int32, (S, BATCH_BLK, VOCAB), 2)
          == t3[:, :, None]).astype(jnp.float32)
    x_tok = lax.dot_general(oh, emb_ref[...],
                            (((2,), (0,)), ((), ())),
                            preferred_element_type=jnp.float32)
    hbp = lax.dot_general(hb_ref[...], pw_ref[...],
                          (((2,), (1,)), ((), ())),
                          preferred_element_type=jnp.float32)
    x = x_tok + hbp * bs_ref[0, 0]
    g = jax.nn.sigmoid(g_ref[...])
    xprev = jnp.concatenate(
        [jnp.zeros((1, BATCH_BLK, D_MODEL), jnp.float32), x[:-1]], axis=0)
    x = (1.0 - g) * x + g * xprev
    ms = jnp.mean(x * x, axis=2, keepdims=True)
    xn = x * lax.rsqrt(ms + 1e-6) * ns_ref[...]
    for s in range(S):
        lt = lax.dot_general(emb_ref[...], xn[s],
                             (((1,), (1,)), ((), ())),
                             preferred_element_type=jnp.float32)
        out_ref[s] = lt


def _tc_dense(tokens_t, hb3, tok_emb, proj_w, big_scale, gate, norm_scale):
    grid = (B // BATCH_BLK,)
    return pl.pallas_call(
        _tc_body,
        grid=grid,
        in_specs=[
            pl.BlockSpec((S, BATCH_BLK), lambda i: (0, i)),
            pl.BlockSpec((S, BATCH_BLK, BIGRAM_DIM), lambda i: (0, i, 0)),
            pl.BlockSpec((VOCAB, D_MODEL), lambda i: (0, 0)),
            pl.BlockSpec((D_MODEL, BIGRAM_DIM), lambda i: (0, 0)),
            pl.BlockSpec((1, 1), lambda i: (0, 0)),
            pl.BlockSpec((1, 1, D_MODEL), lambda i: (0, 0, 0)),
            pl.BlockSpec((1, 1, D_MODEL), lambda i: (0, 0, 0)),
        ],
        out_specs=pl.BlockSpec((S, VOCAB, BATCH_BLK), lambda i: (0, 0, i)),
        out_shape=jax.ShapeDtypeStruct((S, VOCAB, B), jnp.float32),
    )(tokens_t, hb3, tok_emb, proj_w, big_scale, gate, norm_scale)


def kernel(tokens, tok_emb, big_emb, proj_w, big_scale, gate, norm_scale):
    tokens_t = tokens.T.astype(jnp.int32)
    hb = _sc_big(tokens_t, big_emb)
    hb3 = hb.reshape(S, B, BIGRAM_DIM)
    out_t = _tc_dense(tokens_t, hb3, tok_emb, proj_w,
                      big_scale.reshape(1, 1).astype(jnp.float32),
                      gate.reshape(1, 1, D_MODEL),
                      norm_scale.reshape(1, 1, D_MODEL))
    return out_t.transpose(2, 0, 1)
